# baseline (speedup 1.0000x reference)
import functools

import jax
import jax.numpy as jnp
from jax import lax
from jax.experimental import pallas as pl
from jax.experimental.pallas import tpu as pltpu
from jax.experimental.pallas import tpu_sc as plsc

D = 128
N = 320000
NC = 2
NS = 16
NW = NC * NS
L = 16
NVEC = D // L

N_SC = 128000
CH = 400
RPW = N_SC // NW
NCHUNK = RPW // CH

TCCH = 2400
NBUF = 8
N_TC = N - N_SC
NCH_TC = N_TC // TCCH

_mesh = plsc.VectorSubcoreMesh(core_axis_name="c", subcore_axis_name="s")


@functools.partial(
    pl.kernel,
    out_type=jax.ShapeDtypeStruct((NW * D,), jnp.float32),
    mesh=_mesh,
    scratch_types=[
        pltpu.VMEM((CH, D), jnp.float32),
        pltpu.VMEM((CH, D), jnp.float32),
        pltpu.VMEM((D,), jnp.float32),
        pltpu.SemaphoreType.DMA,
        pltpu.SemaphoreType.DMA,
    ],
)
def _sc_partial_sums(nbr_hbm, out_hbm, buf0, buf1, accv, sem0, sem1):
    wid = lax.axis_index("s") * NC + lax.axis_index("c")
    base = wid * RPW

    def start(chunk_idx, buf, sem):
        pltpu.async_copy(nbr_hbm.at[pl.ds(base + chunk_idx * CH, CH)], buf, sem)

    def wait(buf, sem):
        pltpu.make_async_copy(nbr_hbm.at[pl.ds(0, CH)], buf, sem).wait()

    def accum(buf, acc):
        def row_body(r, a):
            return tuple(a[v] + buf[r, pl.ds(v * L, L)] for v in range(NVEC))
        return lax.fori_loop(0, CH, row_body, acc, unroll=2)

    start(0, buf0, sem0)
    start(1, buf1, sem1)

    acc0 = tuple(jnp.zeros((L,), jnp.float32) for _ in range(NVEC))

    def outer(i, acc):
        wait(buf0, sem0)
        acc = accum(buf0, acc)

        @pl.when(i < NCHUNK // 2 - 1)
        def _():
            start(2 * i + 2, buf0, sem0)

        wait(buf1, sem1)
        acc = accum(buf1, acc)

        @pl.when(i < NCHUNK // 2 - 1)
        def _():
            start(2 * i + 3, buf1, sem1)

        return acc

    acc = lax.fori_loop(0, NCHUNK // 2, outer, acc0)

    for v in range(NVEC):
        accv[pl.ds(v * L, L)] = acc[v]
    pltpu.sync_copy(accv, out_hbm.at[pl.ds(wid * D, D)])


def _tc_reduce_kernel(x_hbm, o_ref, buf, sems):
    def start(chunk_idx, slot):
        pltpu.make_async_copy(
            x_hbm.at[pl.ds(N_SC + chunk_idx * TCCH, TCCH)],
            buf.at[slot],
            sems.at[slot],
        ).start()

    for s in range(NBUF):
        start(s, s)

    KCH = 4
    Q = TCCH // KCH

    def body(g, accs):
        slot = lax.rem(g, NBUF)
        pltpu.make_async_copy(
            x_hbm.at[pl.ds(N_SC, TCCH)], buf.at[slot], sems.at[slot]
        ).wait()
        accs = tuple(
            accs[k]
            + jnp.sum(buf[slot, pl.ds(k * Q, Q), :].reshape(Q // 8, 8, D), axis=0)
            for k in range(KCH)
        )

        @pl.when(g + NBUF < NCH_TC)
        def _():
            start(g + NBUF, slot)

        return accs

    accs0 = tuple(jnp.zeros((8, D), jnp.float32) for _ in range(KCH))
    accs = lax.fori_loop(0, NCH_TC, body, accs0)
    total = accs[0]
    for k in range(1, KCH):
        total += accs[k]
    o_ref[...] = total


def _tc_finish_kernel(p_ref, q_ref, c_ref, w_ref, b_ref, o_ref):
    agg = (jnp.sum(p_ref[...], axis=0, keepdims=True)
           + jnp.sum(q_ref[...], axis=0, keepdims=True) + c_ref[...])
    prod = lax.dot_general(
        agg, w_ref[...], (((1,), (1,)), ((), ())),
        preferred_element_type=jnp.float32,
    )
    o_ref[...] = jnp.maximum(prod + b_ref[...], 0.0)


def kernel(neighbor_embs, central_emb, W, b):
    sc_partials = _sc_partial_sums(neighbor_embs).reshape(NW, D)
    tc_partial = pl.pallas_call(
        _tc_reduce_kernel,
        in_specs=[pl.BlockSpec(memory_space=pl.ANY)],
        out_shape=jax.ShapeDtypeStruct((8, D), jnp.float32),
        scratch_shapes=[
            pltpu.VMEM((NBUF, TCCH, D), jnp.float32),
            pltpu.SemaphoreType.DMA((NBUF,)),
        ],
    )(neighbor_embs)
    out = pl.pallas_call(
        _tc_finish_kernel,
        out_shape=jax.ShapeDtypeStruct((1, D), jnp.float32),
    )(sc_partials, tc_partial, central_emb.reshape(1, D), W, b.reshape(1, D))
    return out[0]


# speedup vs baseline: 1.0158x; 1.0158x over previous
import functools

import jax
import jax.numpy as jnp
from jax import lax
from jax.experimental import pallas as pl
from jax.experimental.pallas import tpu as pltpu
from jax.experimental.pallas import tpu_sc as plsc

D = 128
N = 320000
NC = 2
NS = 16
NW = NC * NS
L = 16
NVEC = D // L

N_SC = 115200
CH = 200
RPW = N_SC // NW
NCHUNK = RPW // CH

TCCH = 2560
NBUF = 8
N_TC = N - N_SC
NCH_TC = N_TC // TCCH

_mesh = plsc.VectorSubcoreMesh(core_axis_name="c", subcore_axis_name="s")


@functools.partial(
    pl.kernel,
    out_type=jax.ShapeDtypeStruct((NW * D,), jnp.float32),
    mesh=_mesh,
    scratch_types=[
        pltpu.VMEM((CH, D), jnp.float32),
        pltpu.VMEM((CH, D), jnp.float32),
        pltpu.VMEM((D,), jnp.float32),
        pltpu.SemaphoreType.DMA,
        pltpu.SemaphoreType.DMA,
    ],
)
def _sc_partial_sums(nbr_hbm, out_hbm, buf0, buf1, accv, sem0, sem1):
    wid = lax.axis_index("s") * NC + lax.axis_index("c")
    base = wid * RPW

    def start(chunk_idx, buf, sem):
        pltpu.async_copy(nbr_hbm.at[pl.ds(base + chunk_idx * CH, CH)], buf, sem)

    def wait(buf, sem):
        pltpu.make_async_copy(nbr_hbm.at[pl.ds(0, CH)], buf, sem).wait()

    def accum(buf, acc):
        def row_body(r, a):
            return tuple(a[v] + buf[r, pl.ds(v * L, L)] for v in range(NVEC))
        return lax.fori_loop(0, CH, row_body, acc, unroll=2)

    start(0, buf0, sem0)
    start(1, buf1, sem1)

    acc0 = tuple(jnp.zeros((L,), jnp.float32) for _ in range(NVEC))

    def outer(i, acc):
        wait(buf0, sem0)
        acc = accum(buf0, acc)

        @pl.when(i < NCHUNK // 2 - 1)
        def _():
            start(2 * i + 2, buf0, sem0)

        wait(buf1, sem1)
        acc = accum(buf1, acc)

        @pl.when(i < NCHUNK // 2 - 1)
        def _():
            start(2 * i + 3, buf1, sem1)

        return acc

    acc = lax.fori_loop(0, NCHUNK // 2, outer, acc0)

    for v in range(NVEC):
        accv[pl.ds(v * L, L)] = acc[v]
    pltpu.sync_copy(accv, out_hbm.at[pl.ds(wid * D, D)])


def _tc_reduce_kernel(x_hbm, o_ref, buf, sems):
    def start(chunk_idx, slot):
        pltpu.make_async_copy(
            x_hbm.at[pl.ds(N_SC + chunk_idx * TCCH, TCCH)],
            buf.at[slot],
            sems.at[slot],
        ).start()

    for s in range(NBUF):
        start(s, s)

    KCH = 4
    Q = TCCH // KCH

    def body(g, accs):
        slot = lax.rem(g, NBUF)
        pltpu.make_async_copy(
            x_hbm.at[pl.ds(N_SC, TCCH)], buf.at[slot], sems.at[slot]
        ).wait()
        accs = tuple(
            accs[k]
            + jnp.sum(buf[slot, pl.ds(k * Q, Q), :].reshape(Q // 8, 8, D), axis=0)
            for k in range(KCH)
        )

        @pl.when(g + NBUF < NCH_TC)
        def _():
            start(g + NBUF, slot)

        return accs

    accs0 = tuple(jnp.zeros((8, D), jnp.float32) for _ in range(KCH))
    accs = lax.fori_loop(0, NCH_TC, body, accs0)
    total = accs[0]
    for k in range(1, KCH):
        total += accs[k]
    o_ref[...] = total


def _tc_finish_kernel(p_ref, q_ref, c_ref, w_ref, b_ref, o_ref):
    agg = (jnp.sum(p_ref[...], axis=0, keepdims=True)
           + jnp.sum(q_ref[...], axis=0, keepdims=True) + c_ref[...])
    prod = lax.dot_general(
        agg, w_ref[...], (((1,), (1,)), ((), ())),
        preferred_element_type=jnp.float32,
    )
    o_ref[...] = jnp.maximum(prod + b_ref[...], 0.0)


def kernel(neighbor_embs, central_emb, W, b):
    sc_partials = _sc_partial_sums(neighbor_embs).reshape(NW, D)
    tc_partial = pl.pallas_call(
        _tc_reduce_kernel,
        in_specs=[pl.BlockSpec(memory_space=pl.ANY)],
        out_shape=jax.ShapeDtypeStruct((8, D), jnp.float32),
        scratch_shapes=[
            pltpu.VMEM((NBUF, TCCH, D), jnp.float32),
            pltpu.SemaphoreType.DMA((NBUF,)),
        ],
    )(neighbor_embs)
    out = pl.pallas_call(
        _tc_finish_kernel,
        out_shape=jax.ShapeDtypeStruct((1, D), jnp.float32),
    )(sc_partials, tc_partial, central_emb.reshape(1, D), W, b.reshape(1, D))
    return out[0]


# speedup vs baseline: 1.0184x; 1.0026x over previous
import functools

import jax
import jax.numpy as jnp
from jax import lax
from jax.experimental import pallas as pl
from jax.experimental.pallas import tpu as pltpu
from jax.experimental.pallas import tpu_sc as plsc

D = 128
N = 320000
NC = 2
NS = 16
NW = NC * NS
L = 16
NVEC = D // L

N_SC = 102400
CH = 200
RPW = N_SC // NW
NCHUNK = RPW // CH

TCCH = 2720
NBUF = 8
N_TC = N - N_SC
NCH_TC = N_TC // TCCH

_mesh = plsc.VectorSubcoreMesh(core_axis_name="c", subcore_axis_name="s")


@functools.partial(
    pl.kernel,
    out_type=jax.ShapeDtypeStruct((NW * D,), jnp.float32),
    mesh=_mesh,
    scratch_types=[
        pltpu.VMEM((CH, D), jnp.float32),
        pltpu.VMEM((CH, D), jnp.float32),
        pltpu.VMEM((D,), jnp.float32),
        pltpu.SemaphoreType.DMA,
        pltpu.SemaphoreType.DMA,
    ],
)
def _sc_partial_sums(nbr_hbm, out_hbm, buf0, buf1, accv, sem0, sem1):
    wid = lax.axis_index("s") * NC + lax.axis_index("c")
    base = wid * RPW

    def start(chunk_idx, buf, sem):
        pltpu.async_copy(nbr_hbm.at[pl.ds(base + chunk_idx * CH, CH)], buf, sem)

    def wait(buf, sem):
        pltpu.make_async_copy(nbr_hbm.at[pl.ds(0, CH)], buf, sem).wait()

    def accum(buf, acc):
        def row_body(r, a):
            return tuple(a[v] + buf[r, pl.ds(v * L, L)] for v in range(NVEC))
        return lax.fori_loop(0, CH, row_body, acc, unroll=2)

    start(0, buf0, sem0)
    start(1, buf1, sem1)

    acc0 = tuple(jnp.zeros((L,), jnp.float32) for _ in range(NVEC))

    def outer(i, acc):
        wait(buf0, sem0)
        acc = accum(buf0, acc)

        @pl.when(i < NCHUNK // 2 - 1)
        def _():
            start(2 * i + 2, buf0, sem0)

        wait(buf1, sem1)
        acc = accum(buf1, acc)

        @pl.when(i < NCHUNK // 2 - 1)
        def _():
            start(2 * i + 3, buf1, sem1)

        return acc

    acc = lax.fori_loop(0, NCHUNK // 2, outer, acc0)

    for v in range(NVEC):
        accv[pl.ds(v * L, L)] = acc[v]
    pltpu.sync_copy(accv, out_hbm.at[pl.ds(wid * D, D)])


def _tc_reduce_kernel(x_hbm, o_ref, buf, sems):
    def start(chunk_idx, slot):
        pltpu.make_async_copy(
            x_hbm.at[pl.ds(N_SC + chunk_idx * TCCH, TCCH)],
            buf.at[slot],
            sems.at[slot],
        ).start()

    for s in range(NBUF):
        start(s, s)

    KCH = 4
    Q = TCCH // KCH

    def body(g, accs):
        slot = lax.rem(g, NBUF)
        pltpu.make_async_copy(
            x_hbm.at[pl.ds(N_SC, TCCH)], buf.at[slot], sems.at[slot]
        ).wait()
        accs = tuple(
            accs[k]
            + jnp.sum(buf[slot, pl.ds(k * Q, Q), :].reshape(Q // 8, 8, D), axis=0)
            for k in range(KCH)
        )

        @pl.when(g + NBUF < NCH_TC)
        def _():
            start(g + NBUF, slot)

        return accs

    accs0 = tuple(jnp.zeros((8, D), jnp.float32) for _ in range(KCH))
    accs = lax.fori_loop(0, NCH_TC, body, accs0)
    total = accs[0]
    for k in range(1, KCH):
        total += accs[k]
    o_ref[...] = total


def _tc_finish_kernel(p_ref, q_ref, c_ref, w_ref, b_ref, o_ref):
    agg = (jnp.sum(p_ref[...], axis=0, keepdims=True)
           + jnp.sum(q_ref[...], axis=0, keepdims=True) + c_ref[...])
    prod = lax.dot_general(
        agg, w_ref[...], (((1,), (1,)), ((), ())),
        preferred_element_type=jnp.float32,
    )
    o_ref[...] = jnp.maximum(prod + b_ref[...], 0.0)


def kernel(neighbor_embs, central_emb, W, b):
    sc_partials = _sc_partial_sums(neighbor_embs).reshape(NW, D)
    tc_partial = pl.pallas_call(
        _tc_reduce_kernel,
        in_specs=[pl.BlockSpec(memory_space=pl.ANY)],
        out_shape=jax.ShapeDtypeStruct((8, D), jnp.float32),
        scratch_shapes=[
            pltpu.VMEM((NBUF, TCCH, D), jnp.float32),
            pltpu.SemaphoreType.DMA((NBUF,)),
        ],
    )(neighbor_embs)
    out = pl.pallas_call(
        _tc_finish_kernel,
        out_shape=jax.ShapeDtypeStruct((1, D), jnp.float32),
    )(sc_partials, tc_partial, central_emb.reshape(1, D), W, b.reshape(1, D))
    return out[0]
